# baseline (speedup 1.0000x reference)
import jax
import jax.numpy as jnp
from jax.experimental import pallas as pl
from jax.experimental.pallas import tpu as pltpu

_BETA = 0.8
_LAMBDA_P = 1.0
_LAMBDA_N = 0.1

_IMGS = 16
_ROWS = 512
_COLS = 512
_N_TOTAL = _IMGS * _ROWS * _COLS
_BLK = 8
_STEPS = _IMGS // _BLK


def _red_body(r_ref, lab_ref, o_ref, acc_ref):
    i = pl.program_id(0)
    r = r_ref[...].reshape(_BLK * _ROWS, _COLS)
    lab = lab_ref[...].reshape(_BLK * _ROWS, _COLS)
    p = (lab != 0).astype(jnp.float32)
    rp = r * p
    r2 = r * r
    r2p = r2 * p
    blk = jnp.concatenate(
        [
            jnp.sum(p, axis=0, keepdims=True),
            jnp.sum(rp, axis=0, keepdims=True),
            jnp.sum(r2p, axis=0, keepdims=True),
            jnp.sum(r, axis=0, keepdims=True),
            jnp.sum(r2, axis=0, keepdims=True),
        ],
        axis=0,
    )

    @pl.when(i == 0)
    def _init():
        acc_ref[...] = blk

    @pl.when(i != 0)
    def _accum():
        acc_ref[...] = acc_ref[...] + blk

    @pl.when(i == _STEPS - 1)
    def _finish():
        a = acc_ref[...]
        n = jnp.float32(_N_TOTAL)
        n_p = jnp.sum(a[0:1, :])
        s_p = jnp.sum(a[1:2, :])
        ss_p = jnp.sum(a[2:3, :])
        s_a = jnp.sum(a[3:4, :])
        ss_a = jnp.sum(a[4:5, :])

        n_n = n - n_p
        s_n = s_a - s_p
        ss_n = ss_a - ss_p

        mean_p = s_p / n_p
        var_p = (ss_p - s_p * mean_p) / (n_p - 1.0)
        mean_n = s_n / n_n
        var_n = (ss_n - s_n * (s_n / n_n)) / (n_n - 1.0)

        loss = jnp.maximum(_BETA - mean_p, 0.0)
        loss = loss + _LAMBDA_N * var_p
        loss = loss + mean_n
        loss = loss + _LAMBDA_P * var_n
        o_ref[0] = loss


_reduce = pl.pallas_call(
    _red_body,
    grid=(_STEPS,),
    in_specs=[
        pl.BlockSpec((_BLK, 1, _ROWS, _COLS), lambda i: (i, 0, 0, 0)),
        pl.BlockSpec((_BLK, 1, _ROWS, _COLS), lambda i: (i, 0, 0, 0)),
    ],
    out_specs=pl.BlockSpec(memory_space=pltpu.SMEM),
    out_shape=jax.ShapeDtypeStruct((1,), jnp.float32),
    scratch_shapes=[pltpu.VMEM((5, _COLS), jnp.float32)],
)


def kernel(residues, pixel_level_labels):
    return _reduce(residues, pixel_level_labels)


# speedup vs baseline: 1.0971x; 1.0971x over previous
import jax
import jax.numpy as jnp
from jax.experimental import pallas as pl
from jax.experimental.pallas import tpu as pltpu

_BETA = 0.8
_LAMBDA_P = 1.0
_LAMBDA_N = 0.1

_IMGS = 16
_ROWS = 512
_COLS = 512
_N_TOTAL = _IMGS * _ROWS * _COLS
_BLK = 4
_STEPS = _IMGS // _BLK


def _red_body(r_ref, lab_ref, o_ref, acc_ref):
    i = pl.program_id(0)
    r = r_ref[...].reshape(_BLK * _ROWS, _COLS)
    lab = lab_ref[...].reshape(_BLK * _ROWS, _COLS)
    p = (lab != 0).astype(jnp.float32)
    rp = r * p
    r2 = r * r
    r2p = r2 * p
    blk = jnp.concatenate(
        [
            jnp.sum(p, axis=0, keepdims=True),
            jnp.sum(rp, axis=0, keepdims=True),
            jnp.sum(r2p, axis=0, keepdims=True),
            jnp.sum(r, axis=0, keepdims=True),
            jnp.sum(r2, axis=0, keepdims=True),
        ],
        axis=0,
    )

    @pl.when(i == 0)
    def _init():
        acc_ref[...] = blk

    @pl.when(i != 0)
    def _accum():
        acc_ref[...] = acc_ref[...] + blk

    @pl.when(i == _STEPS - 1)
    def _finish():
        a = acc_ref[...]
        n = jnp.float32(_N_TOTAL)
        n_p = jnp.sum(a[0:1, :])
        s_p = jnp.sum(a[1:2, :])
        ss_p = jnp.sum(a[2:3, :])
        s_a = jnp.sum(a[3:4, :])
        ss_a = jnp.sum(a[4:5, :])

        n_n = n - n_p
        s_n = s_a - s_p
        ss_n = ss_a - ss_p

        mean_p = s_p / n_p
        var_p = (ss_p - s_p * mean_p) / (n_p - 1.0)
        mean_n = s_n / n_n
        var_n = (ss_n - s_n * (s_n / n_n)) / (n_n - 1.0)

        loss = jnp.maximum(_BETA - mean_p, 0.0)
        loss = loss + _LAMBDA_N * var_p
        loss = loss + mean_n
        loss = loss + _LAMBDA_P * var_n
        o_ref[0] = loss


_reduce = pl.pallas_call(
    _red_body,
    grid=(_STEPS,),
    in_specs=[
        pl.BlockSpec((_BLK, 1, _ROWS, _COLS), lambda i: (i, 0, 0, 0)),
        pl.BlockSpec((_BLK, 1, _ROWS, _COLS), lambda i: (i, 0, 0, 0)),
    ],
    out_specs=pl.BlockSpec(memory_space=pltpu.SMEM),
    out_shape=jax.ShapeDtypeStruct((1,), jnp.float32),
    scratch_shapes=[pltpu.VMEM((5, _COLS), jnp.float32)],
)


def kernel(residues, pixel_level_labels):
    return _reduce(residues, pixel_level_labels)
